# Initial kernel scaffold; baseline (speedup 1.0000x reference)
import functools

import jax
import jax.numpy as jnp
from jax import lax
from jax.experimental import pallas as pl
from jax.experimental.pallas import tpu as pltpu
from jax.experimental.pallas import tpu_sc as plsc

M = 100000
B = 16384
D = 256
L = 16
NC = 2
NS = 16
NW = NC * NS
TS = 3136
NV_FULL = TS // L
TAIL = M - (NW - 1) * TS
NV_TAIL = TAIL // L
CHUNK_A = 112
TRIPS_A = TS // CHUNK_A
CHUNK_B = 96
TRIPS_B = TAIL // CHUNK_B
WC = 64
POS_BITS = 14
POS_MASK = (1 << POS_BITS) - 1


def _body(ids_hbm, msgs_hbm, ts_hbm, mem_hbm,
          newmem_hbm, upd_hbm, aggts_hbm,
          ids_v, ts_v, table_v, upd_loc, tsl_loc, winp_v, wins_v,
          key16, srcidx_v, dstidx_v, rows_v, cbuf_v,
          gsem, ssem):
  wid = lax.axis_index("c") * NS + lax.axis_index("s")
  lo = wid * TS
  hi = lo + TS
  size = jnp.minimum(TS, M - lo)
  nv = size // L
  iota = lax.iota(jnp.int32, L)

  pltpu.sync_copy(ids_hbm, ids_v)
  pltpu.sync_copy(ts_hbm, ts_v)

  def init(i, _):
    table_v[pl.ds(i * L, L)] = jnp.full((L,), -1, jnp.int32)
    return 0
  lax.fori_loop(0, NV_FULL, init, 0)

  def scan(i, _):
    ids = ids_v[pl.ds(i * L, L)]
    inr = (ids >= lo) & (ids < hi)
    cnt = jnp.sum(inr.astype(jnp.int32))

    @pl.when(cnt > 0)
    def _():
      poss = i * L + iota
      key = (ids << POS_BITS) + poss
      skey = jnp.sort(key)
      sid = skey >> POS_BITS
      spos = skey & POS_MASK
      key16[...] = sid
      nxt = plsc.load_gather(key16, [jnp.minimum(iota + 1, L - 1)])
      is_last = (sid != nxt) | (iota == L - 1)
      msk = is_last & (sid >= lo) & (sid < hi)
      idx = jnp.clip(sid - lo, 0, TS - 1)
      plsc.store_scatter(table_v, [idx], spos, mask=msk)
    return 0
  lax.fori_loop(0, B // L, scan, 0)

  def emit(j, k):
    lp = table_v[pl.ds(j * L, L)]
    upd = lp >= 0
    updi = upd.astype(jnp.int32)
    safe = jnp.maximum(lp, 0)
    tsg = plsc.load_gather(ts_v, [safe]) * updi.astype(jnp.float32)
    upd_loc[pl.ds(j * L, L)] = updi
    tsl_loc[pl.ds(j * L, L)] = tsg
    csum = plsc.cumsum(updi)
    offs = jnp.clip(k + csum - 1, 0, TS - 1)
    plsc.store_scatter(winp_v, [offs], safe, mask=upd)
    slot = lo + j * L + iota
    plsc.store_scatter(wins_v, [offs], slot, mask=upd)
    return k + jnp.sum(updi)
  k = lax.fori_loop(0, nv, emit, jnp.int32(0))

  def out_full():
    pltpu.sync_copy(upd_loc, upd_hbm.at[pl.ds(lo, TS)])
    pltpu.sync_copy(tsl_loc, aggts_hbm.at[pl.ds(lo, TS)])
  def out_tail():
    pltpu.sync_copy(upd_loc.at[pl.ds(0, TAIL)], upd_hbm.at[pl.ds(lo, TAIL)])
    pltpu.sync_copy(tsl_loc.at[pl.ds(0, TAIL)], aggts_hbm.at[pl.ds(lo, TAIL)])
  lax.cond(nv == NV_FULL, out_full, out_tail)

  def copy_full():
    def trip(t, _):
      base = lo + t * CHUNK_A
      pltpu.sync_copy(mem_hbm.at[pl.ds(base, CHUNK_A), :], cbuf_v)
      pltpu.sync_copy(cbuf_v, newmem_hbm.at[pl.ds(base, CHUNK_A), :])
      return 0
    lax.fori_loop(0, TRIPS_A, trip, 0)
  def copy_tail():
    def trip(t, _):
      base = lo + t * CHUNK_B
      pltpu.sync_copy(mem_hbm.at[pl.ds(base, CHUNK_B), :],
                      cbuf_v.at[pl.ds(0, CHUNK_B), :])
      pltpu.sync_copy(cbuf_v.at[pl.ds(0, CHUNK_B), :],
                      newmem_hbm.at[pl.ds(base, CHUNK_B), :])
      return 0
    lax.fori_loop(0, TRIPS_B, trip, 0)
  lax.cond(nv == NV_FULL, copy_full, copy_tail)

  @pl.when(k > 0)
  def _():
    nch = (k + WC - 1) // WC
    kpad = nch * WC
    klast = jnp.full((L,), 0, jnp.int32) + (k - 1)
    lastw = plsc.load_gather(winp_v, [klast])
    lasts = plsc.load_gather(wins_v, [klast])

    def pad(t, _):
      idx = k + t * L + iota
      m = idx < kpad
      ii = jnp.clip(idx, 0, TS - 1)
      plsc.store_scatter(winp_v, [ii], lastw, mask=m)
      plsc.store_scatter(wins_v, [ii], lasts, mask=m)
      return 0
    lax.fori_loop(0, WC // L, pad, 0)

    def chunk(c, _):
      def ld(t, _):
        gidx = c * WC + t * L + iota
        srcidx_v[pl.ds(t * L, L)] = plsc.load_gather(winp_v, [gidx])
        dstidx_v[pl.ds(t * L, L)] = plsc.load_gather(wins_v, [gidx])
        return 0
      lax.fori_loop(0, WC // L, ld, 0)
      pltpu.async_copy(msgs_hbm.at[srcidx_v], rows_v, gsem).wait()
      pltpu.async_copy(rows_v, newmem_hbm.at[dstidx_v], ssem).wait()
      return 0
    lax.fori_loop(0, nch, chunk, 0)


@functools.partial(
    pl.kernel,
    out_type=[
        jax.ShapeDtypeStruct((M, D), jnp.float32),
        jax.ShapeDtypeStruct((M,), jnp.int32),
        jax.ShapeDtypeStruct((M,), jnp.float32),
    ],
    mesh=plsc.VectorSubcoreMesh(core_axis_name="c", subcore_axis_name="s"),
    scratch_types=[
        pltpu.VMEM((B,), jnp.int32),
        pltpu.VMEM((B,), jnp.float32),
        pltpu.VMEM((TS,), jnp.int32),
        pltpu.VMEM((TS,), jnp.int32),
        pltpu.VMEM((TS,), jnp.float32),
        pltpu.VMEM((TS,), jnp.int32),
        pltpu.VMEM((TS,), jnp.int32),
        pltpu.VMEM((L,), jnp.int32),
        pltpu.VMEM((WC,), jnp.int32),
        pltpu.VMEM((WC,), jnp.int32),
        pltpu.VMEM((WC, D), jnp.float32),
        pltpu.VMEM((CHUNK_A, D), jnp.float32),
        pltpu.SemaphoreType.DMA,
        pltpu.SemaphoreType.DMA,
    ],
)
def _sc_aggregate(ids_hbm, msgs_hbm, ts_hbm, mem_hbm,
                  newmem_hbm, upd_hbm, aggts_hbm, *scratch):
  _body(ids_hbm, msgs_hbm, ts_hbm, mem_hbm,
        newmem_hbm, upd_hbm, aggts_hbm, *scratch)


def kernel(node_ids, messages, timestamps, mem):
  node_ids = node_ids.astype(jnp.int32)
  timestamps = timestamps.astype(jnp.float32)
  new_mem, upd, agg_ts = _sc_aggregate(node_ids, messages, timestamps, mem)
  return new_mem, upd.astype(bool), agg_ts


# speedup vs baseline: 6.0488x; 6.0488x over previous
import functools

import jax
import jax.numpy as jnp
from jax import lax
from jax.experimental import pallas as pl
from jax.experimental.pallas import tpu as pltpu
from jax.experimental.pallas import tpu_sc as plsc

M = 100000
B = 16384
D = 256
L = 16
NC = 2
NS = 16
NW = NC * NS
TS = 3136
NV_FULL = TS // L
TAIL = M - (NW - 1) * TS
CHUNK_A = 112
TRIPS_A = TS // CHUNK_A
CHUNK_B = 96
TRIPS_B = TAIL // CHUNK_B
WC = 64
WC_SHIFT = 6
NCH_MAX = TS // WC
POS_BITS = 14
POS_MASK = (1 << POS_BITS) - 1


def _body(ids_hbm, msgs_hbm, ts_hbm, mem_hbm,
          newmem_hbm, upd_hbm, aggts_hbm,
          ids_v, ts_v, table_v, upd_loc, tsl_loc, winp_v, wins_v,
          key16, srcidx_v, dstidx_v, rows_v, cbuf_v,
          gsem, ssem):
  wid = lax.axis_index("c") * NS + lax.axis_index("s")
  lo = wid * TS
  hi = lo + TS
  size = jnp.minimum(TS, M - lo)
  nv = size // L
  iota = lax.iota(jnp.int32, L)

  pltpu.sync_copy(ids_hbm, ids_v)
  pltpu.sync_copy(ts_hbm, ts_v)

  def init(i, _):
    table_v[pl.ds(i * L, L)] = jnp.full((L,), -1, jnp.int32)
    return 0
  lax.fori_loop(0, NV_FULL, init, 0)

  def scan(i, _):
    ids = ids_v[pl.ds(i * L, L)]
    inr = (ids >= lo) & (ids < hi)

    @pl.when(jnp.any(inr))
    def _():
      poss = i * L + iota
      key = (ids << POS_BITS) + poss
      skey = jnp.sort(key)
      sid = skey >> POS_BITS
      spos = skey & POS_MASK
      key16[...] = sid
      nxt = plsc.load_gather(key16, [jnp.minimum(iota + 1, L - 1)])
      is_last = (sid != nxt) | (iota == L - 1)
      msk = is_last & (sid >= lo) & (sid < hi)
      idx = jnp.clip(sid - lo, 0, TS - 1)
      plsc.store_scatter(table_v, [idx], spos, mask=msk)
    return 0
  lax.fori_loop(0, B // L, scan, 0)

  def emit(j, kvec):
    lp = table_v[pl.ds(j * L, L)]
    upd = lp >= 0
    updi = upd.astype(jnp.int32)
    safe = jnp.maximum(lp, 0)
    tsg = plsc.load_gather(ts_v, [safe]) * updi.astype(jnp.float32)
    upd_loc[pl.ds(j * L, L)] = updi
    tsl_loc[pl.ds(j * L, L)] = tsg
    csum = plsc.cumsum(updi)
    offs = jnp.clip(kvec + csum - 1, 0, TS - 1)
    plsc.store_scatter(winp_v, [offs], safe, mask=upd)
    slot = lo + j * L + iota
    plsc.store_scatter(wins_v, [offs], slot, mask=upd)
    return kvec + plsc.all_reduce_population_count(upd)
  kvec = lax.fori_loop(0, nv, emit, jnp.zeros((L,), jnp.int32))

  def out_full():
    pltpu.sync_copy(upd_loc, upd_hbm.at[pl.ds(lo, TS)])
    pltpu.sync_copy(tsl_loc, aggts_hbm.at[pl.ds(lo, TS)])
  def out_tail():
    pltpu.sync_copy(upd_loc.at[pl.ds(0, TAIL)], upd_hbm.at[pl.ds(lo, TAIL)])
    pltpu.sync_copy(tsl_loc.at[pl.ds(0, TAIL)], aggts_hbm.at[pl.ds(lo, TAIL)])
  lax.cond(nv == NV_FULL, out_full, out_tail)

  def copy_full():
    def trip(t, _):
      base = lo + t * CHUNK_A
      pltpu.sync_copy(mem_hbm.at[pl.ds(base, CHUNK_A), :], cbuf_v)
      pltpu.sync_copy(cbuf_v, newmem_hbm.at[pl.ds(base, CHUNK_A), :])
      return 0
    lax.fori_loop(0, TRIPS_A, trip, 0)
  def copy_tail():
    def trip(t, _):
      base = lo + t * CHUNK_B
      pltpu.sync_copy(mem_hbm.at[pl.ds(base, CHUNK_B), :],
                      cbuf_v.at[pl.ds(0, CHUNK_B), :])
      pltpu.sync_copy(cbuf_v.at[pl.ds(0, CHUNK_B), :],
                      newmem_hbm.at[pl.ds(base, CHUNK_B), :])
      return 0
    lax.fori_loop(0, TRIPS_B, trip, 0)
  lax.cond(nv == NV_FULL, copy_full, copy_tail)

  @pl.when(jnp.any(kvec > 0))
  def _():
    kpadv = ((kvec + (WC - 1)) >> WC_SHIFT) << WC_SHIFT
    lastw = plsc.load_gather(winp_v, [jnp.maximum(kvec - 1, 0)])
    lasts = plsc.load_gather(wins_v, [jnp.maximum(kvec - 1, 0)])

    def pad(t, _):
      idx = t * L + iota
      m = (idx >= kvec) & (idx < kpadv)
      ii = jnp.clip(idx, 0, TS - 1)
      plsc.store_scatter(winp_v, [ii], lastw, mask=m)
      plsc.store_scatter(wins_v, [ii], lasts, mask=m)
      return 0
    lax.fori_loop(0, NV_FULL, pad, 0)

    def chunk(c, _):
      @pl.when(jnp.any(kvec > c * WC))
      def _():
        def ld(t, _):
          gidx = c * WC + t * L + iota
          srcidx_v[pl.ds(t * L, L)] = plsc.load_gather(winp_v, [gidx])
          dstidx_v[pl.ds(t * L, L)] = plsc.load_gather(wins_v, [gidx])
          return 0
        lax.fori_loop(0, WC // L, ld, 0)
        pltpu.async_copy(msgs_hbm.at[srcidx_v], rows_v, gsem).wait()
        pltpu.async_copy(rows_v, newmem_hbm.at[dstidx_v], ssem).wait()
      return 0
    lax.fori_loop(0, NCH_MAX, chunk, 0)


@functools.partial(
    pl.kernel,
    out_type=[
        jax.ShapeDtypeStruct((M, D), jnp.float32),
        jax.ShapeDtypeStruct((M,), jnp.int32),
        jax.ShapeDtypeStruct((M,), jnp.float32),
    ],
    mesh=plsc.VectorSubcoreMesh(core_axis_name="c", subcore_axis_name="s"),
    compiler_params=pltpu.CompilerParams(needs_layout_passes=False),
    scratch_types=[
        pltpu.VMEM((B,), jnp.int32),
        pltpu.VMEM((B,), jnp.float32),
        pltpu.VMEM((TS,), jnp.int32),
        pltpu.VMEM((TS,), jnp.int32),
        pltpu.VMEM((TS,), jnp.float32),
        pltpu.VMEM((TS,), jnp.int32),
        pltpu.VMEM((TS,), jnp.int32),
        pltpu.VMEM((L,), jnp.int32),
        pltpu.VMEM((WC,), jnp.int32),
        pltpu.VMEM((WC,), jnp.int32),
        pltpu.VMEM((WC, D), jnp.float32),
        pltpu.VMEM((CHUNK_A, D), jnp.float32),
        pltpu.SemaphoreType.DMA,
        pltpu.SemaphoreType.DMA,
    ],
)
def _sc_aggregate(ids_hbm, msgs_hbm, ts_hbm, mem_hbm,
                  newmem_hbm, upd_hbm, aggts_hbm, *scratch):
  _body(ids_hbm, msgs_hbm, ts_hbm, mem_hbm,
        newmem_hbm, upd_hbm, aggts_hbm, *scratch)


def kernel(node_ids, messages, timestamps, mem):
  node_ids = node_ids.astype(jnp.int32)
  timestamps = timestamps.astype(jnp.float32)
  new_mem, upd, agg_ts = _sc_aggregate(node_ids, messages, timestamps, mem)
  return new_mem, upd.astype(bool), agg_ts


# speedup vs baseline: 8.6655x; 1.4326x over previous
import functools

import jax
import jax.numpy as jnp
from jax import lax
from jax.experimental import pallas as pl
from jax.experimental.pallas import tpu as pltpu
from jax.experimental.pallas import tpu_sc as plsc

M = 100000
B = 16384
D = 256
L = 16
NC = 2
NS = 16
NW = NC * NS
TS = 3136
NV_FULL = TS // L
TAIL = M - (NW - 1) * TS
TRIPS_F = 56
CHUNK_F = TS // TRIPS_F
TRIPS_T = 58
CHUNK_T = TAIL // TRIPS_T
NSCAN = B // L
SPH_F = -(-NSCAN // TRIPS_F)
SPH_T = -(-NSCAN // TRIPS_T)
WC = 64
POS_BITS = 14
POS_MASK = (1 << POS_BITS) - 1


def _body(ids_hbm, msgs_hbm, ts_hbm, mem_hbm,
          newmem_hbm, upd_hbm, aggts_hbm,
          ids_v, ts_v, table_v, upd_loc, tsl_loc, winp_v, wins_v,
          key16, srcidx_v, dstidx_v, rows_v, cbufa_v, cbufb_v,
          sem_ids, sem_ts, sem_ina, sem_outa, sem_inb, sem_outb,
          gsem, ssem):
  wid = lax.axis_index("c") * NS + lax.axis_index("s")
  lo = wid * TS
  hi = lo + TS
  size = jnp.minimum(TS, M - lo)
  nv = size // L
  iota = lax.iota(jnp.int32, L)

  pltpu.async_copy(ids_hbm, ids_v, sem_ids)
  pltpu.async_copy(ts_hbm, ts_v, sem_ts)

  def init(i, _):
    table_v[pl.ds(i * L, L)] = jnp.full((L,), -1, jnp.int32)
    return 0
  lax.fori_loop(0, NV_FULL, init, 0)

  pltpu.make_async_copy(ids_hbm, ids_v, sem_ids).wait()

  def scan(i, _):
    ids = ids_v[pl.ds(i * L, L)]
    poss = i * L + iota
    key = (ids << POS_BITS) + poss
    skey = jnp.sort(key)
    sid = skey >> POS_BITS
    spos = skey & POS_MASK
    key16[...] = sid
    nxt = plsc.load_gather(key16, [jnp.minimum(iota + 1, L - 1)])
    is_last = (sid != nxt) | (iota == L - 1)
    msk = is_last & (sid >= lo) & (sid < hi)
    idx = jnp.clip(sid - lo, 0, TS - 1)
    plsc.store_scatter(table_v, [idx], spos, mask=msk)
    return 0

  def copy_scan(chunk, trips, sph, cbufa, cbufb):
    def rd(t, buf, sem):
      pltpu.async_copy(mem_hbm.at[pl.ds(lo + t * chunk, chunk), :], buf, sem)
    def wr(t, buf, sem):
      pltpu.async_copy(buf, newmem_hbm.at[pl.ds(lo + t * chunk, chunk), :], sem)

    rd(0, cbufa, sem_ina)

    def cbody(u, _):
      t0 = 2 * u
      t1 = t0 + 1
      @pl.when(u > 0)
      def _():
        pltpu.make_async_copy(cbufb, newmem_hbm.at[pl.ds(0, chunk), :],
                              sem_outb).wait()
      rd(t1, cbufb, sem_inb)
      base = u * (2 * sph)
      lax.fori_loop(jnp.minimum(base, NSCAN),
                    jnp.minimum(base + sph, NSCAN), scan, 0)
      pltpu.make_async_copy(mem_hbm.at[pl.ds(0, chunk), :], cbufa,
                            sem_ina).wait()
      wr(t0, cbufa, sem_outa)
      lax.fori_loop(jnp.minimum(base + sph, NSCAN),
                    jnp.minimum(base + 2 * sph, NSCAN), scan, 0)
      pltpu.make_async_copy(mem_hbm.at[pl.ds(0, chunk), :], cbufb,
                            sem_inb).wait()
      wr(t1, cbufb, sem_outb)
      @pl.when(u < trips // 2 - 1)
      def _():
        pltpu.make_async_copy(cbufa, newmem_hbm.at[pl.ds(0, chunk), :],
                              sem_outa).wait()
        rd(t0 + 2, cbufa, sem_ina)
      return 0
    lax.fori_loop(0, trips // 2, cbody, 0)
    pltpu.make_async_copy(cbufa, newmem_hbm.at[pl.ds(0, chunk), :],
                          sem_outa).wait()
    pltpu.make_async_copy(cbufb, newmem_hbm.at[pl.ds(0, chunk), :],
                          sem_outb).wait()

  def cs_full():
    copy_scan(CHUNK_F, TRIPS_F, SPH_F, cbufa_v, cbufb_v)
  def cs_tail():
    copy_scan(CHUNK_T, TRIPS_T, SPH_T, cbufa_v.at[pl.ds(0, CHUNK_T), :],
              cbufb_v.at[pl.ds(0, CHUNK_T), :])
  lax.cond(nv == NV_FULL, cs_full, cs_tail)

  pltpu.make_async_copy(ts_hbm, ts_v, sem_ts).wait()
  key16[...] = jnp.zeros((L,), jnp.int32)

  def emit(j, _):
    kvec = key16[...]
    lp = table_v[pl.ds(j * L, L)]
    upd = lp >= 0
    updi = jnp.where(upd, 1, 0)
    safe = jnp.maximum(lp, 0)
    tsg = plsc.load_gather(ts_v, [safe]) * updi.astype(jnp.float32)
    upd_loc[pl.ds(j * L, L)] = updi
    tsl_loc[pl.ds(j * L, L)] = tsg
    csum = plsc.cumsum(updi)
    offs = jnp.clip(kvec + csum - 1, 0, TS - 1)
    plsc.store_scatter(winp_v, [offs], safe, mask=upd)
    slot = lo + j * L + iota
    plsc.store_scatter(wins_v, [offs], slot, mask=upd)
    key16[...] = kvec + plsc.all_reduce_population_count(upd)
    return 0
  lax.fori_loop(0, nv, emit, 0)
  k_s = key16[...][0]

  def out_full():
    pltpu.sync_copy(upd_loc, upd_hbm.at[pl.ds(lo, TS)])
    pltpu.sync_copy(tsl_loc, aggts_hbm.at[pl.ds(lo, TS)])
  def out_tail():
    pltpu.sync_copy(upd_loc.at[pl.ds(0, TAIL)], upd_hbm.at[pl.ds(lo, TAIL)])
    pltpu.sync_copy(tsl_loc.at[pl.ds(0, TAIL)], aggts_hbm.at[pl.ds(lo, TAIL)])
  lax.cond(nv == NV_FULL, out_full, out_tail)

  @pl.when(k_s > 0)
  def _():
    nch = (k_s + WC - 1) // WC
    kpad = nch * WC
    klast = jnp.full((L,), 0, jnp.int32) + (k_s - 1)
    lastw = plsc.load_gather(winp_v, [klast])
    lasts = plsc.load_gather(wins_v, [klast])

    def pad(t, _):
      idx = k_s + t * L + iota
      m = idx < kpad
      ii = jnp.clip(idx, 0, TS - 1)
      plsc.store_scatter(winp_v, [ii], lastw, mask=m)
      plsc.store_scatter(wins_v, [ii], lasts, mask=m)
      return 0
    lax.fori_loop(0, WC // L, pad, 0)

    def chunk(c, _):
      def ld(t, _):
        gidx = c * WC + t * L + iota
        srcidx_v[pl.ds(t * L, L)] = plsc.load_gather(winp_v, [gidx])
        dstidx_v[pl.ds(t * L, L)] = plsc.load_gather(wins_v, [gidx])
        return 0
      lax.fori_loop(0, WC // L, ld, 0)
      pltpu.async_copy(msgs_hbm.at[srcidx_v], rows_v, gsem).wait()
      pltpu.async_copy(rows_v, newmem_hbm.at[dstidx_v], ssem).wait()
      return 0
    lax.fori_loop(0, nch, chunk, 0)


@functools.partial(
    pl.kernel,
    out_type=[
        jax.ShapeDtypeStruct((M, D), jnp.float32),
        jax.ShapeDtypeStruct((M,), jnp.int32),
        jax.ShapeDtypeStruct((M,), jnp.float32),
    ],
    mesh=plsc.VectorSubcoreMesh(core_axis_name="c", subcore_axis_name="s"),
    compiler_params=pltpu.CompilerParams(needs_layout_passes=False),
    scratch_types=[
        pltpu.VMEM((B,), jnp.int32),
        pltpu.VMEM((B,), jnp.float32),
        pltpu.VMEM((TS,), jnp.int32),
        pltpu.VMEM((TS,), jnp.int32),
        pltpu.VMEM((TS,), jnp.float32),
        pltpu.VMEM((TS,), jnp.int32),
        pltpu.VMEM((TS,), jnp.int32),
        pltpu.VMEM((L,), jnp.int32),
        pltpu.VMEM((WC,), jnp.int32),
        pltpu.VMEM((WC,), jnp.int32),
        pltpu.VMEM((WC, D), jnp.float32),
        pltpu.VMEM((CHUNK_F, D), jnp.float32),
        pltpu.VMEM((CHUNK_F, D), jnp.float32),
        pltpu.SemaphoreType.DMA,
        pltpu.SemaphoreType.DMA,
        pltpu.SemaphoreType.DMA,
        pltpu.SemaphoreType.DMA,
        pltpu.SemaphoreType.DMA,
        pltpu.SemaphoreType.DMA,
        pltpu.SemaphoreType.DMA,
        pltpu.SemaphoreType.DMA,
    ],
)
def _sc_aggregate(ids_hbm, msgs_hbm, ts_hbm, mem_hbm,
                  newmem_hbm, upd_hbm, aggts_hbm, *scratch):
  _body(ids_hbm, msgs_hbm, ts_hbm, mem_hbm,
        newmem_hbm, upd_hbm, aggts_hbm, *scratch)


def kernel(node_ids, messages, timestamps, mem):
  node_ids = node_ids.astype(jnp.int32)
  timestamps = timestamps.astype(jnp.float32)
  new_mem, upd, agg_ts = _sc_aggregate(node_ids, messages, timestamps, mem)
  return new_mem, upd.astype(bool), agg_ts


# speedup vs baseline: 9.4350x; 1.0888x over previous
import functools

import jax
import jax.numpy as jnp
from jax import lax
from jax.experimental import pallas as pl
from jax.experimental.pallas import tpu as pltpu
from jax.experimental.pallas import tpu_sc as plsc

M = 100000
B = 16384
D = 256
L = 16
NC = 2
NS = 16
NW = NC * NS
TS = 3136
NV_FULL = TS // L
TAIL = M - (NW - 1) * TS
NBUF = 4
TRIPS_F = 56
CHUNK_F = TS // TRIPS_F
TRIPS_T = 116
CHUNK_T = TAIL // TRIPS_T
NSCAN = B // L
SPH_F = -(-NSCAN // TRIPS_F)
SPH_T = -(-NSCAN // TRIPS_T)
WC = 32
POS_BITS = 14
POS_MASK = (1 << POS_BITS) - 1


def _body(ids_hbm, msgs_hbm, ts_hbm, mem_hbm,
          newmem_hbm, upd_hbm, aggts_hbm,
          ids_v, ts_v, table_v, upd_loc, tsl_loc, winp_v, wins_v,
          key16, src0_v, dst0_v, src1_v, dst1_v, rows0_v, rows1_v,
          cb0, cb1, cb2, cb3,
          sem_ids, sem_ts, in0, in1, in2, in3, out0, out1, out2, out3,
          g0, s0, g1, s1):
  wid = lax.axis_index("c") * NS + lax.axis_index("s")
  lo = wid * TS
  hi = lo + TS
  size = jnp.minimum(TS, M - lo)
  nv = size // L
  iota = lax.iota(jnp.int32, L)
  bufs = [cb0, cb1, cb2, cb3]
  isems = [in0, in1, in2, in3]
  osems = [out0, out1, out2, out3]

  pltpu.async_copy(ids_hbm, ids_v, sem_ids)
  pltpu.async_copy(ts_hbm, ts_v, sem_ts)

  def init(i, _):
    table_v[pl.ds(i * L, L)] = jnp.full((L,), -1, jnp.int32)
    return 0
  lax.fori_loop(0, NV_FULL, init, 0)

  pltpu.make_async_copy(ids_hbm, ids_v, sem_ids).wait()

  def scan(i, _):
    ids = ids_v[pl.ds(i * L, L)]
    poss = i * L + iota
    key = (ids << POS_BITS) + poss
    skey = jnp.sort(key)
    sid = skey >> POS_BITS
    spos = skey & POS_MASK
    key16[...] = sid
    nxt = plsc.load_gather(key16, [jnp.minimum(iota + 1, L - 1)])
    is_last = (sid != nxt) | (iota == L - 1)
    msk = is_last & (sid >= lo) & (sid < hi)
    idx = jnp.clip(sid - lo, 0, TS - 1)
    plsc.store_scatter(table_v, [idx], spos, mask=msk)
    return 0

  def copy_scan(chunk, trips, sph, bslices):
    def rd(t, buf, sem):
      pltpu.async_copy(mem_hbm.at[pl.ds(lo + t * chunk, chunk), :], buf, sem)
    def wr(t, buf, sem):
      pltpu.async_copy(buf, newmem_hbm.at[pl.ds(lo + t * chunk, chunk), :], sem)

    for i in range(NBUF):
      rd(i, bslices[i], isems[i])

    def cbody(u, _):
      tb = NBUF * u
      for i in range(NBUF):
        t = tb + i
        pltpu.make_async_copy(mem_hbm.at[pl.ds(0, chunk), :], bslices[i],
                              isems[i]).wait()
        wr(t, bslices[i], osems[i])
        seg = tb + i
        lax.fori_loop(jnp.minimum(seg * sph, NSCAN),
                      jnp.minimum((seg + 1) * sph, NSCAN), scan, 0)
        @pl.when(t + NBUF < trips)
        def _():
          pltpu.make_async_copy(bslices[i], newmem_hbm.at[pl.ds(0, chunk), :],
                                osems[i]).wait()
          rd(t + NBUF, bslices[i], isems[i])
      return 0
    lax.fori_loop(0, trips // NBUF, cbody, 0)
    for i in range(NBUF):
      pltpu.make_async_copy(bslices[i], newmem_hbm.at[pl.ds(0, chunk), :],
                            osems[i]).wait()

  def cs_full():
    copy_scan(CHUNK_F, TRIPS_F, SPH_F, bufs)
  def cs_tail():
    copy_scan(CHUNK_T, TRIPS_T, SPH_T,
              [b.at[pl.ds(0, CHUNK_T), :] for b in bufs])
  lax.cond(nv == NV_FULL, cs_full, cs_tail)

  pltpu.make_async_copy(ts_hbm, ts_v, sem_ts).wait()
  key16[...] = jnp.zeros((L,), jnp.int32)

  def emit(j, _):
    kvec = key16[...]
    lp = table_v[pl.ds(j * L, L)]
    upd = lp >= 0
    updi = jnp.where(upd, 1, 0)
    safe = jnp.maximum(lp, 0)
    tsg = plsc.load_gather(ts_v, [safe]) * updi.astype(jnp.float32)
    upd_loc[pl.ds(j * L, L)] = updi
    tsl_loc[pl.ds(j * L, L)] = tsg
    csum = plsc.cumsum(updi)
    offs = jnp.clip(kvec + csum - 1, 0, TS - 1)
    plsc.store_scatter(winp_v, [offs], safe, mask=upd)
    slot = lo + j * L + iota
    plsc.store_scatter(wins_v, [offs], slot, mask=upd)
    key16[...] = kvec + plsc.all_reduce_population_count(upd)
    return 0
  lax.fori_loop(0, nv, emit, 0)
  k_s = key16[...][0]

  def out_start_full():
    pltpu.async_copy(upd_loc, upd_hbm.at[pl.ds(lo, TS)], sem_ids)
    pltpu.async_copy(tsl_loc, aggts_hbm.at[pl.ds(lo, TS)], sem_ts)
  def out_start_tail():
    pltpu.async_copy(upd_loc.at[pl.ds(0, TAIL)],
                     upd_hbm.at[pl.ds(lo, TAIL)], sem_ids)
    pltpu.async_copy(tsl_loc.at[pl.ds(0, TAIL)],
                     aggts_hbm.at[pl.ds(lo, TAIL)], sem_ts)
  lax.cond(nv == NV_FULL, out_start_full, out_start_tail)

  @pl.when(k_s > 0)
  def _():
    nch = (k_s + WC - 1) // WC
    kpad = nch * WC
    klast = jnp.full((L,), 0, jnp.int32) + (k_s - 1)
    lastw = plsc.load_gather(winp_v, [klast])
    lasts = plsc.load_gather(wins_v, [klast])

    def pad(t, _):
      idx = k_s + t * L + iota
      m = idx < kpad
      ii = jnp.clip(idx, 0, TS - 1)
      plsc.store_scatter(winp_v, [ii], lastw, mask=m)
      plsc.store_scatter(wins_v, [ii], lasts, mask=m)
      return 0
    lax.fori_loop(0, WC // L, pad, 0)

    def ldidx(c, sref, dref):
      def ld(t, _):
        gidx = c * WC + t * L + iota
        sref[pl.ds(t * L, L)] = plsc.load_gather(winp_v, [gidx])
        dref[pl.ds(t * L, L)] = plsc.load_gather(wins_v, [gidx])
        return 0
      lax.fori_loop(0, WC // L, ld, 0)

    ldidx(0, src0_v, dst0_v)
    pltpu.async_copy(msgs_hbm.at[src0_v], rows0_v, g0)

    def wbody(u, _):
      c0 = 2 * u
      c1 = c0 + 1
      @pl.when(c1 < nch)
      def _():
        @pl.when(u > 0)
        def _():
          pltpu.make_async_copy(rows1_v, newmem_hbm.at[dst1_v], s1).wait()
        ldidx(c1, src1_v, dst1_v)
        pltpu.async_copy(msgs_hbm.at[src1_v], rows1_v, g1)
      pltpu.make_async_copy(msgs_hbm.at[src0_v], rows0_v, g0).wait()
      pltpu.async_copy(rows0_v, newmem_hbm.at[dst0_v], s0)
      @pl.when(c0 + 2 < nch)
      def _():
        pltpu.make_async_copy(rows0_v, newmem_hbm.at[dst0_v], s0).wait()
        ldidx(c0 + 2, src0_v, dst0_v)
        pltpu.async_copy(msgs_hbm.at[src0_v], rows0_v, g0)
      @pl.when(c1 < nch)
      def _():
        pltpu.make_async_copy(msgs_hbm.at[src1_v], rows1_v, g1).wait()
        pltpu.async_copy(rows1_v, newmem_hbm.at[dst1_v], s1)
      return 0
    lax.fori_loop(0, (nch + 1) // 2, wbody, 0)
    pltpu.make_async_copy(rows0_v, newmem_hbm.at[dst0_v], s0).wait()
    @pl.when(nch > 1)
    def _():
      pltpu.make_async_copy(rows1_v, newmem_hbm.at[dst1_v], s1).wait()

  def out_wait_full():
    pltpu.make_async_copy(upd_loc, upd_hbm.at[pl.ds(lo, TS)], sem_ids).wait()
    pltpu.make_async_copy(tsl_loc, aggts_hbm.at[pl.ds(lo, TS)], sem_ts).wait()
  def out_wait_tail():
    pltpu.make_async_copy(upd_loc.at[pl.ds(0, TAIL)],
                          upd_hbm.at[pl.ds(lo, TAIL)], sem_ids).wait()
    pltpu.make_async_copy(tsl_loc.at[pl.ds(0, TAIL)],
                          aggts_hbm.at[pl.ds(lo, TAIL)], sem_ts).wait()
  lax.cond(nv == NV_FULL, out_wait_full, out_wait_tail)


@functools.partial(
    pl.kernel,
    out_type=[
        jax.ShapeDtypeStruct((M, D), jnp.float32),
        jax.ShapeDtypeStruct((M,), jnp.int32),
        jax.ShapeDtypeStruct((M,), jnp.float32),
    ],
    mesh=plsc.VectorSubcoreMesh(core_axis_name="c", subcore_axis_name="s"),
    compiler_params=pltpu.CompilerParams(needs_layout_passes=False),
    scratch_types=[
        pltpu.VMEM((B,), jnp.int32),
        pltpu.VMEM((B,), jnp.float32),
        pltpu.VMEM((TS,), jnp.int32),
        pltpu.VMEM((TS,), jnp.int32),
        pltpu.VMEM((TS,), jnp.float32),
        pltpu.VMEM((TS,), jnp.int32),
        pltpu.VMEM((TS,), jnp.int32),
        pltpu.VMEM((L,), jnp.int32),
        pltpu.VMEM((WC,), jnp.int32),
        pltpu.VMEM((WC,), jnp.int32),
        pltpu.VMEM((WC,), jnp.int32),
        pltpu.VMEM((WC,), jnp.int32),
        pltpu.VMEM((WC, D), jnp.float32),
        pltpu.VMEM((WC, D), jnp.float32),
        pltpu.VMEM((CHUNK_F, D), jnp.float32),
        pltpu.VMEM((CHUNK_F, D), jnp.float32),
        pltpu.VMEM((CHUNK_F, D), jnp.float32),
        pltpu.VMEM((CHUNK_F, D), jnp.float32),
        pltpu.SemaphoreType.DMA,
        pltpu.SemaphoreType.DMA,
        pltpu.SemaphoreType.DMA,
        pltpu.SemaphoreType.DMA,
        pltpu.SemaphoreType.DMA,
        pltpu.SemaphoreType.DMA,
        pltpu.SemaphoreType.DMA,
        pltpu.SemaphoreType.DMA,
        pltpu.SemaphoreType.DMA,
        pltpu.SemaphoreType.DMA,
        pltpu.SemaphoreType.DMA,
        pltpu.SemaphoreType.DMA,
        pltpu.SemaphoreType.DMA,
        pltpu.SemaphoreType.DMA,
    ],
)
def _sc_aggregate(ids_hbm, msgs_hbm, ts_hbm, mem_hbm,
                  newmem_hbm, upd_hbm, aggts_hbm, *scratch):
  _body(ids_hbm, msgs_hbm, ts_hbm, mem_hbm,
        newmem_hbm, upd_hbm, aggts_hbm, *scratch)


def kernel(node_ids, messages, timestamps, mem):
  node_ids = node_ids.astype(jnp.int32)
  timestamps = timestamps.astype(jnp.float32)
  new_mem, upd, agg_ts = _sc_aggregate(node_ids, messages, timestamps, mem)
  return new_mem, upd.astype(bool), agg_ts


# speedup vs baseline: 10.4854x; 1.1113x over previous
import functools

import jax
import jax.numpy as jnp
from jax import lax
from jax.experimental import pallas as pl
from jax.experimental.pallas import tpu as pltpu
from jax.experimental.pallas import tpu_sc as plsc

M = 100000
B = 16384
D = 256
L = 16
NC = 2
NS = 16
NW = NC * NS
TS = 3136
NV_FULL = TS // L
TAIL = M - (NW - 1) * TS
NSCAN = B // L
WC = 32
POS_BITS = 14
POS_MASK = (1 << POS_BITS) - 1
CBLK = 10000


def _cp_body(x_ref, o_ref):
  o_ref[...] = x_ref[...]


_tc_copy = pl.pallas_call(
    _cp_body,
    grid=(M // CBLK,),
    in_specs=[pl.BlockSpec((CBLK, D), lambda i: (i, 0))],
    out_specs=pl.BlockSpec((CBLK, D), lambda i: (i, 0)),
    out_shape=jax.ShapeDtypeStruct((M, D), jnp.float32),
)


def _prep_body(ids_hbm, ts_hbm,
               upd_hbm, aggts_hbm, winp_hbm, wins_hbm, cnt_hbm,
               ids_v, ts_v, table_v, upd_loc, tsl_loc, winp_v, wins_v,
               key16, sem_ids, sem_ts):
  wid = lax.axis_index("c") * NS + lax.axis_index("s")
  lo = wid * TS
  hi = lo + TS
  size = jnp.minimum(TS, M - lo)
  nv = size // L
  iota = lax.iota(jnp.int32, L)

  pltpu.async_copy(ids_hbm, ids_v, sem_ids)
  pltpu.async_copy(ts_hbm, ts_v, sem_ts)

  def init(i, _):
    table_v[pl.ds(i * L, L)] = jnp.full((L,), -1, jnp.int32)
    return 0
  lax.fori_loop(0, NV_FULL, init, 0)

  pltpu.make_async_copy(ids_hbm, ids_v, sem_ids).wait()

  def scan(i, _):
    ids = ids_v[pl.ds(i * L, L)]
    inr = (ids >= lo) & (ids < hi)

    @pl.when(jnp.any(inr))
    def _():
      poss = i * L + iota
      key = (ids << POS_BITS) + poss
      skey = jnp.sort(key)
      sid = skey >> POS_BITS
      spos = skey & POS_MASK
      key16[...] = sid
      nxt = plsc.load_gather(key16, [jnp.minimum(iota + 1, L - 1)])
      is_last = (sid != nxt) | (iota == L - 1)
      msk = is_last & (sid >= lo) & (sid < hi)
      idx = jnp.clip(sid - lo, 0, TS - 1)
      plsc.store_scatter(table_v, [idx], spos, mask=msk)
    return 0
  lax.fori_loop(0, NSCAN, scan, 0)

  pltpu.make_async_copy(ts_hbm, ts_v, sem_ts).wait()
  key16[...] = jnp.zeros((L,), jnp.int32)

  def emit(j, _):
    kvec = key16[...]
    lp = table_v[pl.ds(j * L, L)]
    upd = lp >= 0
    updi = jnp.where(upd, 1, 0)
    safe = jnp.maximum(lp, 0)
    tsg = plsc.load_gather(ts_v, [safe]) * updi.astype(jnp.float32)
    upd_loc[pl.ds(j * L, L)] = updi
    tsl_loc[pl.ds(j * L, L)] = tsg
    csum = plsc.cumsum(updi)
    offs = jnp.clip(kvec + csum - 1, 0, TS - 1)
    plsc.store_scatter(winp_v, [offs], safe, mask=upd)
    slot = lo + j * L + iota
    plsc.store_scatter(wins_v, [offs], slot, mask=upd)
    key16[...] = kvec + plsc.all_reduce_population_count(upd)
    return 0
  lax.fori_loop(0, nv, emit, 0)
  kvec = key16[...]
  k_s = kvec[0]

  @pl.when(k_s > 0)
  def _():
    kpad = ((k_s + WC - 1) // WC) * WC
    klast = jnp.full((L,), 0, jnp.int32) + (k_s - 1)
    lastw = plsc.load_gather(winp_v, [klast])
    lasts = plsc.load_gather(wins_v, [klast])

    def pad(t, _):
      idx = k_s + t * L + iota
      m = idx < kpad
      ii = jnp.clip(idx, 0, TS - 1)
      plsc.store_scatter(winp_v, [ii], lastw, mask=m)
      plsc.store_scatter(wins_v, [ii], lasts, mask=m)
      return 0
    lax.fori_loop(0, WC // L, pad, 0)

  pltpu.async_copy(winp_v, winp_hbm.at[wid], sem_ids)
  pltpu.async_copy(wins_v, wins_hbm.at[wid], sem_ts)

  def out_full():
    pltpu.sync_copy(upd_loc, upd_hbm.at[pl.ds(lo, TS)])
    pltpu.sync_copy(tsl_loc, aggts_hbm.at[pl.ds(lo, TS)])
  def out_tail():
    pltpu.sync_copy(upd_loc.at[pl.ds(0, TAIL)], upd_hbm.at[pl.ds(lo, TAIL)])
    pltpu.sync_copy(tsl_loc.at[pl.ds(0, TAIL)], aggts_hbm.at[pl.ds(lo, TAIL)])
  lax.cond(nv == NV_FULL, out_full, out_tail)

  key16[...] = kvec
  pltpu.sync_copy(key16, cnt_hbm.at[wid])
  pltpu.make_async_copy(winp_v, winp_hbm.at[wid], sem_ids).wait()
  pltpu.make_async_copy(wins_v, wins_hbm.at[wid], sem_ts).wait()


_sc_prep = pl.kernel(
    _prep_body,
    out_type=[
        jax.ShapeDtypeStruct((M,), jnp.int32),
        jax.ShapeDtypeStruct((M,), jnp.float32),
        jax.ShapeDtypeStruct((NW, TS), jnp.int32),
        jax.ShapeDtypeStruct((NW, TS), jnp.int32),
        jax.ShapeDtypeStruct((NW, L), jnp.int32),
    ],
    mesh=plsc.VectorSubcoreMesh(core_axis_name="c", subcore_axis_name="s"),
    compiler_params=pltpu.CompilerParams(needs_layout_passes=False),
    scratch_types=[
        pltpu.VMEM((B,), jnp.int32),
        pltpu.VMEM((B,), jnp.float32),
        pltpu.VMEM((TS,), jnp.int32),
        pltpu.VMEM((TS,), jnp.int32),
        pltpu.VMEM((TS,), jnp.float32),
        pltpu.VMEM((TS,), jnp.int32),
        pltpu.VMEM((TS,), jnp.int32),
        pltpu.VMEM((L,), jnp.int32),
        pltpu.SemaphoreType.DMA,
        pltpu.SemaphoreType.DMA,
    ],
)


def _scat_body(msgs_hbm, winp_hbm, wins_hbm, cnt_hbm, newmem_hbm,
               winp_v, wins_v, key16, src0_v, dst0_v, src1_v, dst1_v,
               rows0_v, rows1_v, g0, s0, g1, s1):
  wid = lax.axis_index("c") * NS + lax.axis_index("s")
  pltpu.async_copy(winp_hbm.at[wid], winp_v, g0)
  pltpu.async_copy(wins_hbm.at[wid], wins_v, g1)
  pltpu.sync_copy(cnt_hbm.at[wid], key16)
  k_s = key16[...][0]
  pltpu.make_async_copy(winp_hbm.at[wid], winp_v, g0).wait()
  pltpu.make_async_copy(wins_hbm.at[wid], wins_v, g1).wait()

  @pl.when(k_s > 0)
  def _():
    nch = (k_s + WC - 1) // WC

    def ldidx(c, sref, dref):
      def ld(t, _):
        gidx = c * WC + t * L + lax.iota(jnp.int32, L)
        sref[pl.ds(t * L, L)] = plsc.load_gather(winp_v, [gidx])
        dref[pl.ds(t * L, L)] = plsc.load_gather(wins_v, [gidx])
        return 0
      lax.fori_loop(0, WC // L, ld, 0)

    ldidx(0, src0_v, dst0_v)
    pltpu.async_copy(msgs_hbm.at[src0_v], rows0_v, g0)

    def wbody(u, _):
      c0 = 2 * u
      c1 = c0 + 1
      @pl.when(c1 < nch)
      def _():
        @pl.when(u > 0)
        def _():
          pltpu.make_async_copy(rows1_v, newmem_hbm.at[dst1_v], s1).wait()
        ldidx(c1, src1_v, dst1_v)
        pltpu.async_copy(msgs_hbm.at[src1_v], rows1_v, g1)
      pltpu.make_async_copy(msgs_hbm.at[src0_v], rows0_v, g0).wait()
      pltpu.async_copy(rows0_v, newmem_hbm.at[dst0_v], s0)
      @pl.when(c0 + 2 < nch)
      def _():
        pltpu.make_async_copy(rows0_v, newmem_hbm.at[dst0_v], s0).wait()
        ldidx(c0 + 2, src0_v, dst0_v)
        pltpu.async_copy(msgs_hbm.at[src0_v], rows0_v, g0)
      @pl.when(c1 < nch)
      def _():
        pltpu.make_async_copy(msgs_hbm.at[src1_v], rows1_v, g1).wait()
        pltpu.async_copy(rows1_v, newmem_hbm.at[dst1_v], s1)
      return 0
    lax.fori_loop(0, (nch + 1) // 2, wbody, 0)
    pltpu.make_async_copy(rows0_v, newmem_hbm.at[dst0_v], s0).wait()
    @pl.when(nch > 1)
    def _():
      pltpu.make_async_copy(rows1_v, newmem_hbm.at[dst1_v], s1).wait()


_sc_scatter = pl.kernel(
    _scat_body,
    out_type=[],
    mesh=plsc.VectorSubcoreMesh(core_axis_name="c", subcore_axis_name="s"),
    compiler_params=pltpu.CompilerParams(needs_layout_passes=False),
    scratch_types=[
        pltpu.VMEM((TS,), jnp.int32),
        pltpu.VMEM((TS,), jnp.int32),
        pltpu.VMEM((L,), jnp.int32),
        pltpu.VMEM((WC,), jnp.int32),
        pltpu.VMEM((WC,), jnp.int32),
        pltpu.VMEM((WC,), jnp.int32),
        pltpu.VMEM((WC,), jnp.int32),
        pltpu.VMEM((WC, D), jnp.float32),
        pltpu.VMEM((WC, D), jnp.float32),
        pltpu.SemaphoreType.DMA,
        pltpu.SemaphoreType.DMA,
        pltpu.SemaphoreType.DMA,
        pltpu.SemaphoreType.DMA,
    ],
)


def kernel(node_ids, messages, timestamps, mem):
  node_ids = node_ids.astype(jnp.int32)
  timestamps = timestamps.astype(jnp.float32)
  new_mem0 = _tc_copy(mem)
  upd, agg_ts, winp, wins, cnt = _sc_prep(node_ids, timestamps)
  r = jax.new_ref(new_mem0)
  _sc_scatter(messages, winp, wins, cnt, r)
  return r[...], upd.astype(bool), agg_ts
